# Initial kernel scaffold; baseline (speedup 1.0000x reference)
import jax
import jax.numpy as jnp
from jax import lax
from jax.experimental import pallas as pl
from jax.experimental.pallas import tpu as pltpu

H = 1024
I = 512
E = 64
NG = 8
GS = E // NG
TOPK_GROUP = 4
TOP_K = 8
N_SHARED = 2
N_TOK = 512

NEG = jnp.float32(-1e30)


def _first_max_mask(work, axis):
    m = jnp.max(work, axis=axis, keepdims=True)
    ismax = work == m
    idx = lax.broadcasted_iota(jnp.int32, work.shape, axis)
    first = jnp.min(jnp.where(ismax, idx, jnp.int32(10**9)), axis=axis,
                    keepdims=True)
    return idx == first


def _gate_combine_T(x, wg):
    lT = lax.dot_general(wg, x, (((1,), (1,)), ((), ())),
                         preferred_element_type=jnp.float32)
    l3 = lT.reshape(NG, GS, N_TOK)
    work = l3
    sel4 = jnp.zeros(l3.shape, dtype=jnp.bool_)
    for _ in range(TOPK_GROUP):
        pick = _first_max_mask(work, 1)
        sel4 = jnp.logical_or(sel4, pick)
        work = jnp.where(pick, NEG, work)
    cand = jnp.where(sel4, l3, NEG).reshape(E, N_TOK)
    sel8 = jnp.zeros(cand.shape, dtype=jnp.bool_)
    work2 = cand
    for _ in range(TOP_K):
        pick = _first_max_mask(work2, 0)
        sel8 = jnp.logical_or(sel8, pick)
        work2 = jnp.where(pick, NEG, work2)
    wsel = jnp.where(sel8, lT, jnp.float32(0.0))
    wsum = jnp.sum(wsel, axis=0, keepdims=True) + jnp.float32(1e-20)
    return wsel / wsum


def _mlp(x, w_gu, w_dn):
    h = jnp.dot(x, w_gu, preferred_element_type=jnp.float32)
    g = h[:, :I]
    u = h[:, I:]
    return jnp.dot(jax.nn.silu(g) * u, w_dn,
                   preferred_element_type=jnp.float32)


def _moe_body(x_ref, wg_ref, wgu_ref, wdn_ref, wsgu_ref, wsdn_ref,
              out_ref, comb_ref):
    e = pl.program_id(0)

    @pl.when(e == 0)
    def _init():
        combT = _gate_combine_T(x_ref[...], wg_ref[...])
        comb_ref[...] = combT.T
        acc = jnp.zeros((N_TOK, H), jnp.float32)
        for s in range(N_SHARED):
            acc = acc + _mlp(x_ref[...], wsgu_ref[s], wsdn_ref[s])
        out_ref[...] = acc

    y = _mlp(x_ref[...], wgu_ref[...], wdn_ref[...])
    onehot = (lax.broadcasted_iota(jnp.int32, (E, 1), 0) == e
              ).astype(jnp.float32)
    col = jnp.dot(comb_ref[...], onehot,
                  preferred_element_type=jnp.float32)
    out_ref[...] += col * y


def kernel(x, Wg, W_gu, W_dn, Ws_gu, Ws_dn):
    return pl.pallas_call(
        _moe_body,
        grid=(E,),
        in_specs=[
            pl.BlockSpec((N_TOK, H), lambda e: (0, 0)),
            pl.BlockSpec((E, H), lambda e: (0, 0)),
            pl.BlockSpec((None, H, 2 * I), lambda e: (e, 0, 0)),
            pl.BlockSpec((None, I, H), lambda e: (e, 0, 0)),
            pl.BlockSpec((N_SHARED, H, 2 * I), lambda e: (0, 0, 0)),
            pl.BlockSpec((N_SHARED, I, H), lambda e: (0, 0, 0)),
        ],
        out_specs=pl.BlockSpec((N_TOK, H), lambda e: (0, 0)),
        out_shape=jax.ShapeDtypeStruct((N_TOK, H), jnp.float32),
        scratch_shapes=[pltpu.VMEM((N_TOK, E), jnp.float32)],
        compiler_params=pltpu.CompilerParams(
            dimension_semantics=("arbitrary",),
        ),
    )(x, Wg, W_gu, W_dn, Ws_gu, Ws_dn)


# speedup vs baseline: 3.9486x; 3.9486x over previous
import jax
import jax.numpy as jnp
from jax import lax
from jax.experimental import pallas as pl
from jax.experimental.pallas import tpu as pltpu

H = 1024
I = 512
E = 64
NG = 8
GS = E // NG
TOPK_GROUP = 4
TOP_K = 8
N_SHARED = 2
N_TOK = 512

NEG = -1e30


def _first_max_mask(work, axis):
    m = jnp.max(work, axis=axis, keepdims=True)
    ismax = work == m
    idx = lax.broadcasted_iota(jnp.int32, work.shape, axis)
    first = jnp.min(jnp.where(ismax, idx, jnp.int32(10**9)), axis=axis,
                    keepdims=True)
    return idx == first


def _gate_combine_T(x, wg):
    lT = lax.dot_general(wg, x, (((1,), (1,)), ((), ())),
                         preferred_element_type=jnp.float32)
    l3 = lT.reshape(NG, GS, N_TOK)
    work = l3
    sel4 = jnp.zeros(l3.shape, dtype=jnp.bool_)
    for _ in range(TOPK_GROUP):
        pick = _first_max_mask(work, 1)
        sel4 = jnp.logical_or(sel4, pick)
        work = jnp.where(pick, NEG, work)
    cand = jnp.where(sel4, l3, NEG).reshape(E, N_TOK)
    sel8 = jnp.zeros(cand.shape, dtype=jnp.bool_)
    work2 = cand
    for _ in range(TOP_K):
        pick = _first_max_mask(work2, 0)
        sel8 = jnp.logical_or(sel8, pick)
        work2 = jnp.where(pick, NEG, work2)
    wsel = jnp.where(sel8, lT, jnp.float32(0.0))
    wsum = jnp.sum(wsel, axis=0, keepdims=True) + jnp.float32(1e-20)
    return wsel / wsum


def _mlp(x, w_gu, w_dn):
    h = jnp.dot(x, w_gu, preferred_element_type=jnp.float32)
    g = h[:, :I]
    u = h[:, I:]
    return jnp.dot(jax.nn.silu(g) * u, w_dn,
                   preferred_element_type=jnp.float32)


def _moe_body(x_ref, wg_ref, wgu_ref, wdn_ref, wsgu_ref, wsdn_ref,
              out_ref, comb_ref):
    e = pl.program_id(0)

    @pl.when(e == 0)
    def _init():
        combT = _gate_combine_T(x_ref[...], wg_ref[...])
        comb_ref[...] = combT.T
        acc = jnp.zeros((N_TOK, H), jnp.float32)
        for s in range(N_SHARED):
            acc = acc + _mlp(x_ref[...], wsgu_ref[s], wsdn_ref[s])
        out_ref[...] = acc

    y = _mlp(x_ref[...], wgu_ref[...], wdn_ref[...])
    onehot = (lax.broadcasted_iota(jnp.int32, (E, 1), 0) == e
              ).astype(jnp.float32)
    col = jnp.dot(comb_ref[...], onehot,
                  preferred_element_type=jnp.float32)
    out_ref[...] += col * y


def kernel(x, Wg, W_gu, W_dn, Ws_gu, Ws_dn):
    return pl.pallas_call(
        _moe_body,
        grid=(E,),
        in_specs=[
            pl.BlockSpec((N_TOK, H), lambda e: (0, 0)),
            pl.BlockSpec((E, H), lambda e: (0, 0)),
            pl.BlockSpec((None, H, 2 * I), lambda e: (e, 0, 0)),
            pl.BlockSpec((None, I, H), lambda e: (e, 0, 0)),
            pl.BlockSpec((N_SHARED, H, 2 * I), lambda e: (0, 0, 0)),
            pl.BlockSpec((N_SHARED, I, H), lambda e: (0, 0, 0)),
        ],
        out_specs=pl.BlockSpec((N_TOK, H), lambda e: (0, 0)),
        out_shape=jax.ShapeDtypeStruct((N_TOK, H), jnp.float32),
        scratch_shapes=[pltpu.VMEM((N_TOK, E), jnp.float32)],
        compiler_params=pltpu.CompilerParams(
            dimension_semantics=("arbitrary",),
        ),
    )(x, Wg, W_gu, W_dn, Ws_gu, Ws_dn)
